# baseline (speedup 1.0000x reference)
import jax
import jax.numpy as jnp
from jax import lax
from jax.experimental import pallas as pl
from jax.experimental.pallas import tpu as pltpu
from jax.experimental.pallas import tpu_sc as plsc

N = 50000
E = 800000
D = 64
P = 2 * D

R_E = 8000
R_N = 2000
RP = R_E // 2
EP = E // 2
CB_EMB = 16000


def _silu(x):
    return x * jax.nn.sigmoid(x)


def _ln_packed(x, g, b, mavg):
    m = jnp.dot(x, mavg, preferred_element_type=jnp.float32)
    d = x - m
    v = jnp.dot(d * d, mavg, preferred_element_type=jnp.float32)
    return d * jax.lax.rsqrt(v + 1e-5) * g + b


def _full(shape):
    return pl.BlockSpec(shape, lambda i: tuple(0 for _ in shape))


def _rows(r, w):
    return pl.BlockSpec((r, w), lambda i: (i, 0))


def _mlp2_body(x_ref, w1_ref, b1_ref, w2_ref, b2_ref, o_ref, ob_ref):
    h = jnp.dot(x_ref[...], w1_ref[...], preferred_element_type=jnp.float32)
    h = _silu(h + b1_ref[...])
    o = jnp.dot(h, w2_ref[...], preferred_element_type=jnp.float32) + b2_ref[...]
    o_ref[...] = o
    ob_ref[...] = o.astype(jnp.bfloat16)


def _mlp2(x, w1, b1, w2, b2, r):
    n = x.shape[0]
    od = w2.shape[1]
    return pl.pallas_call(
        _mlp2_body,
        grid=(n // r,),
        in_specs=[
            _rows(r, x.shape[1]),
            _full(w1.shape), _full(b1.shape), _full(w2.shape), _full(b2.shape),
        ],
        out_specs=[_rows(r, od), _rows(r, od)],
        out_shape=[jax.ShapeDtypeStruct((n, od), jnp.float32),
                   jax.ShapeDtypeStruct((n, od), jnp.bfloat16)],
    )(x, w1, b1, w2, b2)


def _mlp2_pack_body(xa_ref, xb_ref, w1_ref, b1_ref, w2_ref, b2_ref, o_ref):
    def m(xt):
        h = lax.dot_general(xt, w1_ref[...], (((0,), (0,)), ((), ())),
                            preferred_element_type=jnp.float32)
        h = _silu(h + b1_ref[...])
        return jnp.dot(h, w2_ref[...], preferred_element_type=jnp.float32) + b2_ref[...]

    o_ref[...] = jnp.concatenate([m(xa_ref[...]), m(xb_ref[...])], axis=1)


def _mlp2_pack(xt, w1, b1, w2, b2, cb):
    nf = xt.shape[0]
    od = w2.shape[1]
    ng = EP // cb
    a_spec = pl.BlockSpec((nf, cb), lambda i: (0, i))
    b_spec = pl.BlockSpec((nf, cb), lambda i: (0, i + ng))
    return pl.pallas_call(
        _mlp2_pack_body,
        grid=(ng,),
        in_specs=[a_spec, b_spec,
                  _full(w1.shape), _full(b1.shape), _full(w2.shape), _full(b2.shape)],
        out_specs=pl.BlockSpec((cb, 2 * od), lambda i: (i, 0)),
        out_shape=jax.ShapeDtypeStruct((EP, 2 * od), jnp.float32),
    )(xt, xt, w1, b1, w2, b2)


def _msg_body(gs_ref, gd_ref, he_ref, wa_ref, wb_ref, wc_ref, b1_ref,
              w2_ref, b2_ref, o_ref):
    gs = gs_ref[...].astype(jnp.float32)
    gd = gd_ref[...].astype(jnp.float32)
    h = (jnp.dot(gs, wa_ref[...], preferred_element_type=jnp.float32)
         + jnp.dot(gd, wb_ref[...], preferred_element_type=jnp.float32)
         + jnp.dot(he_ref[...], wc_ref[...], preferred_element_type=jnp.float32))
    h = _silu(h + b1_ref[...])
    o_ref[...] = jnp.dot(h, w2_ref[...], preferred_element_type=jnp.float32) + b2_ref[...]


def _g_specs():
    src_spec = pl.BlockSpec((RP, P), lambda i: (i, 0))
    dst_spec = pl.BlockSpec((RP, P), lambda i: (i + E // R_E, 0))
    return src_spec, dst_spec


def _msg(g2, h_edge, mp):
    src_spec, dst_spec = _g_specs()
    return pl.pallas_call(
        _msg_body,
        grid=(E // R_E,),
        in_specs=[src_spec, dst_spec, _rows(RP, P)]
        + [_full(w.shape) for w in mp],
        out_specs=_rows(RP, P),
        out_shape=jax.ShapeDtypeStruct((EP, P), jnp.float32),
    )(g2, g2, h_edge, *mp)


def _node_body(hn_ref, ag_ref, wa_ref, wb_ref, b1_ref, w2_ref, b2_ref,
               g_ref, be_ref, o_ref, ob_ref):
    hn = hn_ref[...]
    h = (jnp.dot(hn, wa_ref[...], preferred_element_type=jnp.float32)
         + jnp.dot(ag_ref[...], wb_ref[...], preferred_element_type=jnp.float32))
    h = _silu(h + b1_ref[...])
    u = jnp.dot(h, w2_ref[...], preferred_element_type=jnp.float32) + b2_ref[...]
    x = hn + u
    m = jnp.mean(x, axis=-1, keepdims=True)
    v = jnp.mean((x - m) ** 2, axis=-1, keepdims=True)
    out = (x - m) * jax.lax.rsqrt(v + 1e-5) * g_ref[...] + be_ref[...]
    o_ref[...] = out
    ob_ref[...] = out.astype(jnp.bfloat16)


def _node_update(h_node, agg, up):
    wa, wb, b1, w2, b2, g, be = up
    return pl.pallas_call(
        _node_body,
        grid=(N // R_N,),
        in_specs=[_rows(R_N, D), _rows(R_N, D),
                  _full(wa.shape), _full(wb.shape), _full(b1.shape),
                  _full(w2.shape), _full(b2.shape), _full(g.shape), _full(be.shape)],
        out_specs=[_rows(R_N, D), _rows(R_N, D)],
        out_shape=[jax.ShapeDtypeStruct((N, D), jnp.float32),
                   jax.ShapeDtypeStruct((N, D), jnp.bfloat16)],
    )(h_node, agg, wa, wb, b1, w2, b2, g, be)


def _edge_stage_body(gs_ref, gd_ref, he_ref, mavg_ref,
                     ea_ref, eb_ref, ec_ref, e1_ref, ew2_ref, e2_ref,
                     lg_ref, lb_ref,
                     ma_ref, mb_ref, mc_ref, m1_ref, mw2_ref, m2_ref,
                     he_out_ref, msg_out_ref):
    gs = gs_ref[...].astype(jnp.float32)
    gd = gd_ref[...].astype(jnp.float32)
    he = he_ref[...]
    h = (jnp.dot(gs, ea_ref[...], preferred_element_type=jnp.float32)
         + jnp.dot(gd, eb_ref[...], preferred_element_type=jnp.float32)
         + jnp.dot(he, ec_ref[...], preferred_element_type=jnp.float32))
    h = _silu(h + e1_ref[...])
    u = jnp.dot(h, ew2_ref[...], preferred_element_type=jnp.float32) + e2_ref[...]
    he_new = _ln_packed(he + u, lg_ref[...], lb_ref[...], mavg_ref[...])
    he_out_ref[...] = he_new
    m = (jnp.dot(gs, ma_ref[...], preferred_element_type=jnp.float32)
         + jnp.dot(gd, mb_ref[...], preferred_element_type=jnp.float32)
         + jnp.dot(he_new, mc_ref[...], preferred_element_type=jnp.float32))
    m = _silu(m + m1_ref[...])
    msg_out_ref[...] = jnp.dot(m, mw2_ref[...], preferred_element_type=jnp.float32) + m2_ref[...]


def _edge_stage(g2, h_edge, mavg, ep, mp):
    src_spec, dst_spec = _g_specs()
    ws = list(ep) + list(mp)
    return pl.pallas_call(
        _edge_stage_body,
        grid=(E // R_E,),
        in_specs=[src_spec, dst_spec, _rows(RP, P), _full(mavg.shape)]
        + [_full(w.shape) for w in ws],
        out_specs=[_rows(RP, P), _rows(RP, P)],
        out_shape=[jax.ShapeDtypeStruct((EP, P), jnp.float32),
                   jax.ShapeDtypeStruct((EP, P), jnp.float32)],
    )(g2, g2, h_edge, mavg, *ws)


def _final_body(gs_ref, gd_ref, he_ref, mavg_ref, sel_ref,
                ea_ref, eb_ref, ec_ref, e1_ref, ew2_ref, e2_ref,
                lg_ref, lb_ref,
                m1a_ref, m1b_ref, m1c_ref, mb1_ref, m2_ref, mb2_ref, m3_ref, mb3_ref,
                r1a_ref, r1b_ref, r1c_ref, rb1_ref, r2_ref, rb2_ref, r3_ref, rb3_ref,
                merge_ref, risk_ref):
    gs = gs_ref[...].astype(jnp.float32)
    gd = gd_ref[...].astype(jnp.float32)
    he = he_ref[...]
    h = (jnp.dot(gs, ea_ref[...], preferred_element_type=jnp.float32)
         + jnp.dot(gd, eb_ref[...], preferred_element_type=jnp.float32)
         + jnp.dot(he, ec_ref[...], preferred_element_type=jnp.float32))
    h = _silu(h + e1_ref[...])
    u = jnp.dot(h, ew2_ref[...], preferred_element_type=jnp.float32) + e2_ref[...]
    he_new = _ln_packed(he + u, lg_ref[...], lb_ref[...], mavg_ref[...])

    def head(w1a, w1b, w1c, b1, w2, b2, w3, b3):
        h1 = (jnp.dot(gs, w1a, preferred_element_type=jnp.float32)
              + jnp.dot(gd, w1b, preferred_element_type=jnp.float32)
              + jnp.dot(he_new, w1c, preferred_element_type=jnp.float32))
        h1 = _silu(h1 + b1)
        h2 = _silu(jnp.dot(h1, w2, preferred_element_type=jnp.float32) + b2)
        return jnp.dot(h2 * w3, sel_ref[...],
                       preferred_element_type=jnp.float32) + b3[0, 0]

    merge_ref[...] = head(m1a_ref[...], m1b_ref[...], m1c_ref[...], mb1_ref[...],
                          m2_ref[...], mb2_ref[...], m3_ref[...], mb3_ref[...])
    risk_ref[...] = jax.nn.sigmoid(
        head(r1a_ref[...], r1b_ref[...], r1c_ref[...], rb1_ref[...],
             r2_ref[...], rb2_ref[...], r3_ref[...], rb3_ref[...]))


def _final_stage(g2, h_edge, mavg, sel, ep, hp_merge, hp_risk):
    src_spec, dst_spec = _g_specs()
    ws = list(ep) + list(hp_merge) + list(hp_risk)
    out_spec = pl.BlockSpec((RP, 2), lambda i: (i, 0))
    merge, risk = pl.pallas_call(
        _final_body,
        grid=(E // R_E,),
        in_specs=[src_spec, dst_spec, _rows(RP, P), _full(mavg.shape),
                  _full(sel.shape)]
        + [_full(w.shape) for w in ws],
        out_specs=[out_spec, out_spec],
        out_shape=[jax.ShapeDtypeStruct((EP, 2), jnp.float32),
                   jax.ShapeDtypeStruct((EP, 2), jnp.float32)],
    )(g2, g2, h_edge, mavg, sel, *ws)
    return (jnp.concatenate([merge[:, 0], merge[:, 1]]),
            jnp.concatenate([risk[:, 0], risk[:, 1]]))


_NC, _NS = 2, 16
_NW = _NC * _NS
_CH = 128
_GCH = 12512
_CPW = _GCH // _NW
_GPAD = _GCH * _CH
_KB = 8


def _gather_body(table, idx3, out, idx_v, *rest):
    bufs = rest[:_KB]
    gsem, wsem = rest[_KB], rest[_KB + 1]
    w = lax.axis_index("s") * _NC + lax.axis_index("c")
    pltpu.sync_copy(idx3.at[w], idx_v)
    base = w * _CPW
    ngrp = (_CPW + _KB - 1) // _KB

    def grp(g, carry):
        for b in range(_KB):
            j = g * _KB + b

            @pl.when(j < _CPW)
            def _():
                @pl.when(g > 0)
                def _():
                    pltpu.make_async_copy(
                        bufs[b], out.at[pl.ds((base + j - _KB) * _CH, _CH)],
                        wsem.at[b]).wait()
                pltpu.async_copy(table.at[idx_v.at[j]], bufs[b], gsem.at[b])
        for b in range(_KB):
            j = g * _KB + b

            @pl.when(j < _CPW)
            def _():
                pltpu.make_async_copy(table.at[idx_v.at[j]], bufs[b],
                                      gsem.at[b]).wait()
                pltpu.async_copy(bufs[b], out.at[pl.ds((base + j) * _CH, _CH)],
                                 wsem.at[b])
        return carry

    lax.fori_loop(0, ngrp, grp, 0)
    ntail = _CPW - (ngrp - 1) * _KB
    for b in range(_KB):
        j = (ngrp - 1) * _KB + b if b < ntail else (ngrp - 2) * _KB + b
        pltpu.make_async_copy(bufs[b], out.at[pl.ds((base + j) * _CH, _CH)],
                              wsem.at[b]).wait()


def _gather(h_node, idx3):
    mesh = plsc.VectorSubcoreMesh(core_axis_name="c", subcore_axis_name="s")
    return pl.kernel(
        _gather_body,
        mesh=mesh,
        compiler_params=pltpu.CompilerParams(use_tc_tiling_on_sc=False),
        out_type=jax.ShapeDtypeStruct((_GPAD, D), jnp.bfloat16),
        scratch_types=(
            [pltpu.VMEM((_CPW, _CH), jnp.int32)]
            + [pltpu.VMEM((_CH, D), jnp.bfloat16) for _ in range(_KB)]
            + [pltpu.SemaphoreType.DMA((_KB,)), pltpu.SemaphoreType.DMA((_KB,))]
        ),
    )(h_node, idx3)


_HQ = D // 4
_SCH = E // _CH
_SCHP = 6256
_CPT = _SCHP // _NS
_AROW = 51200
_ZB = _AROW // _NS // _CH
_KS = 8
_HCH = _CH // 2


def _scatter_body(msg2, dst2, agg, idx_v, zbuf, acc, *rest):
    vbufs = rest[:_KS]
    lsem, ssem = rest[_KS], rest[_KS + 1]
    c = lax.axis_index("c")
    s = lax.axis_index("s")
    base = s * _CPT
    pltpu.sync_copy(dst2.at[pl.ds(base, _CPT)], idx_v)

    def zb(i, carry):
        zbuf[i, pl.ds(0, 16)] = jnp.zeros((16,), jnp.float32)
        return carry

    lax.fori_loop(0, _CH, zb, 0)

    ngrp = (_CPT + _KS - 1) // _KS

    for half in range(2):
        col0 = c * _HQ + half * 2 * _HQ

        def zg(k, carry):
            pltpu.sync_copy(zbuf, acc.at[pl.ds((s * _ZB + k) * _CH, _CH)])
            return carry

        lax.fori_loop(0, _ZB, zg, 0)
        plsc.subcore_barrier()

        def ld(j, b):
            r0 = (base + j) * _HCH
            even = pltpu.async_copy(
                msg2.at[pl.ds(r0, _HCH), pl.ds(col0, _HQ)],
                vbufs[b].at[pl.ds(0, _HCH)], lsem.at[b])
            odd = pltpu.async_copy(
                msg2.at[pl.ds(r0, _HCH), pl.ds(D + col0, _HQ)],
                vbufs[b].at[pl.ds(_HCH, _HCH)], lsem.at[b])
            return even, odd

        def grp(g, carry):
            for b in range(_KS):
                j = g * _KS + b

                @pl.when((j < _CPT) & (base + j < _SCH))
                def _():
                    @pl.when(g > 0)
                    def _():
                        pltpu.make_async_copy(vbufs[b], acc.at[pl.ds(0, _CH)],
                                              ssem.at[b]).wait()
                    ld(j, b)
            for b in range(_KS):
                j = g * _KS + b

                @pl.when((j < _CPT) & (base + j < _SCH))
                def _():
                    r0 = (base + j) * _HCH
                    pltpu.make_async_copy(
                        msg2.at[pl.ds(r0, _HCH), pl.ds(col0, _HQ)],
                        vbufs[b].at[pl.ds(0, _HCH)], lsem.at[b]).wait()
                    pltpu.make_async_copy(
                        msg2.at[pl.ds(r0, _HCH), pl.ds(D + col0, _HQ)],
                        vbufs[b].at[pl.ds(_HCH, _HCH)], lsem.at[b]).wait()
                    pltpu.async_copy(vbufs[b], acc.at[idx_v.at[j]], ssem.at[b],
                                     add=True)
            return carry

        lax.fori_loop(0, ngrp, grp, 0)
        for b in range(_KS):
            last = (_SCH - 1 - base - b) // _KS

            @pl.when(last >= 0)
            def _():
                pltpu.make_async_copy(vbufs[b], acc.at[pl.ds(0, _CH)],
                                      ssem.at[b]).wait()
        plsc.subcore_barrier()
        nr = N // _NS
        pltpu.sync_copy(acc.at[pl.ds(s * nr, nr)],
                        agg.at[pl.ds(s * nr, nr), pl.ds(col0, _HQ)])
        plsc.subcore_barrier()


def _scatter_add(msg2, dst2):
    mesh = plsc.VectorSubcoreMesh(core_axis_name="c", subcore_axis_name="s")
    return pl.kernel(
        _scatter_body,
        mesh=mesh,
        compiler_params=pltpu.CompilerParams(use_tc_tiling_on_sc=False),
        out_type=jax.ShapeDtypeStruct((N, D), jnp.float32),
        scratch_types=(
            [pltpu.VMEM((_CPT, _CH), jnp.int32),
             pltpu.VMEM((_CH, _HQ), jnp.float32),
             pltpu.VMEM_SHARED((_AROW, _HQ), jnp.float32)]
            + [pltpu.VMEM((_CH, _HQ), jnp.float32) for _ in range(_KS)]
            + [pltpu.SemaphoreType.DMA((_KS,)), pltpu.SemaphoreType.DMA((_KS,))]
        ),
    )(msg2, dst2)


def _bd(w):
    return jnp.kron(jnp.eye(2, dtype=jnp.float32), w)


def _t2(b):
    return jnp.tile(b.reshape(1, -1), (1, 2))


def _split3(w):
    return w[:D], w[D:2 * D], w[2 * D:]


def _prep_node_embed(ps):
    (w1, b1), (w2, b2) = ps
    return w1, b1.reshape(1, -1), w2, b2.reshape(1, -1)


def _prep_edge_embed(ps):
    (w1, b1), (w2, b2) = ps
    return w1, b1.reshape(1, -1), w2, b2.reshape(1, -1)


def _prep_msg(ps):
    (w1, b1), (w2, b2) = ps
    wa, wb, wc = _split3(w1)
    return _bd(wa), _bd(wb), _bd(wc), _t2(b1), _bd(w2), _t2(b2)


def _prep_upd(ps, norm):
    (w1, b1), (w2, b2) = ps
    wa, wb = w1[:D], w1[D:]
    g, be = norm
    return wa, wb, b1.reshape(1, -1), w2, b2.reshape(1, -1), g.reshape(1, -1), be.reshape(1, -1)


def _prep_edge(ps, norm):
    (w1, b1), (w2, b2) = ps
    wa, wb, wc = _split3(w1)
    g, be = norm
    return _bd(wa), _bd(wb), _bd(wc), _t2(b1), _bd(w2), _t2(b2), _t2(g), _t2(be)


def _prep_head(ps):
    (w1, b1), (w2, b2), (w3, b3) = ps
    wa, wb, wc = _split3(w1)
    return (_bd(wa), _bd(wb), _bd(wc), _t2(b1), _bd(w2), _t2(b2),
            _t2(w3.reshape(1, -1)), b3.reshape(1, 1))


def kernel(node_feat, edge_index, edge_feat, params):
    src = edge_index[:, 0]
    dst = edge_index[:, 1]
    def _halves(a):
        return jnp.concatenate([a[:EP].reshape(_SCH, _HCH),
                                a[EP:].reshape(_SCH, _HCH)], axis=1)

    src_p = jnp.stack([src[:EP], src[EP:]], axis=1).reshape(E)
    dst_p = jnp.stack([dst[:EP], dst[EP:]], axis=1).reshape(E)
    idx3 = jnp.concatenate(
        [src_p, dst_p, jnp.zeros((_GPAD - 2 * E,), jnp.int32)]).reshape(_NW, _CPW, _CH)
    dst2 = jnp.concatenate(
        [_halves(dst), jnp.zeros((_SCHP - _SCH, _CH), jnp.int32)])

    mavg = _bd(jnp.full((D, D), 1.0 / D, jnp.float32))
    sel = _bd(jnp.ones((D, 1), jnp.float32))

    ne = _prep_node_embed(params["node_embed"])
    ee = _prep_edge_embed(params["edge_embed"])
    layers = [{
        "msg": _prep_msg(lp["msg"]),
        "upd": _prep_upd(lp["upd"], lp["node_norm"]),
        "edge": _prep_edge(lp["edge_upd"], lp["edge_norm"]),
    } for lp in params["layers"]]
    hp_merge = _prep_head(params["merge_head"])
    hp_risk = _prep_head(params["risk_head"])

    h_node, h_bf = _mlp2(node_feat, *ne, R_N)
    h_edge = _mlp2_pack(jnp.swapaxes(edge_feat, 0, 1), *ee, CB_EMB)

    g2 = _gather(h_bf, idx3).reshape(_GPAD // 2, P)
    msg = _msg(g2, h_edge, layers[0]["msg"])
    for i in range(6):
        agg = _scatter_add(msg, dst2)
        h_node, h_bf = _node_update(h_node, agg, layers[i]["upd"])
        g2 = _gather(h_bf, idx3).reshape(_GPAD // 2, P)
        if i < 5:
            h_edge, msg = _edge_stage(g2, h_edge, mavg,
                                      layers[i]["edge"], layers[i + 1]["msg"])
        else:
            merge, risk = _final_stage(g2, h_edge, mavg, sel,
                                       layers[i]["edge"], hp_merge, hp_risk)
    return (merge, risk)


# speedup vs baseline: 1.4766x; 1.4766x over previous
import jax
import jax.numpy as jnp
from jax import lax
from jax.experimental import pallas as pl
from jax.experimental.pallas import tpu as pltpu
from jax.experimental.pallas import tpu_sc as plsc

N = 50000
E = 800000
D = 64
P = 2 * D

R_E = 8000
R_N = 2000
RP = R_E // 2
EP = E // 2
E2 = E // 2
EPH = E2 // 2
CB_EMB = 16000


def _silu(x):
    return x * jax.nn.sigmoid(x)


def _ln_packed(x, g, b, mavg):
    m = jnp.dot(x, mavg, preferred_element_type=jnp.float32)
    d = x - m
    v = jnp.dot(d * d, mavg, preferred_element_type=jnp.float32)
    return d * jax.lax.rsqrt(v + 1e-5) * g + b


def _full(shape):
    return pl.BlockSpec(shape, lambda i: tuple(0 for _ in shape))


def _rows(r, w):
    return pl.BlockSpec((r, w), lambda i: (i, 0))


def _mlp2_body(x_ref, w1_ref, b1_ref, w2_ref, b2_ref, o_ref):
    h = jnp.dot(x_ref[...], w1_ref[...], preferred_element_type=jnp.float32)
    h = _silu(h + b1_ref[...])
    o_ref[...] = jnp.dot(h, w2_ref[...], preferred_element_type=jnp.float32) + b2_ref[...]


def _mlp2(x, w1, b1, w2, b2, r):
    n = x.shape[0]
    od = w2.shape[1]
    return pl.pallas_call(
        _mlp2_body,
        grid=(n // r,),
        in_specs=[
            _rows(r, x.shape[1]),
            _full(w1.shape), _full(b1.shape), _full(w2.shape), _full(b2.shape),
        ],
        out_specs=_rows(r, od),
        out_shape=jax.ShapeDtypeStruct((n, od), jnp.float32),
    )(x, w1, b1, w2, b2)


def _mlp2_pack_body(xa_ref, xb_ref, w1_ref, b1_ref, w2_ref, b2_ref, o_ref):
    def m(xt):
        h = lax.dot_general(xt, w1_ref[...], (((0,), (0,)), ((), ())),
                            preferred_element_type=jnp.float32)
        h = _silu(h + b1_ref[...])
        return jnp.dot(h, w2_ref[...], preferred_element_type=jnp.float32) + b2_ref[...]

    o_ref[...] = jnp.concatenate([m(xa_ref[...]), m(xb_ref[...])], axis=1)


def _mlp2_pack(xt, w1, b1, w2, b2, cb):
    nf = xt.shape[0]
    od = w2.shape[1]
    ng = EP // cb
    a_spec = pl.BlockSpec((nf, cb), lambda i: (0, i))
    b_spec = pl.BlockSpec((nf, cb), lambda i: (0, i + ng))
    return pl.pallas_call(
        _mlp2_pack_body,
        grid=(ng,),
        in_specs=[a_spec, b_spec,
                  _full(w1.shape), _full(b1.shape), _full(w2.shape), _full(b2.shape)],
        out_specs=pl.BlockSpec((cb, 2 * od), lambda i: (i, 0)),
        out_shape=jax.ShapeDtypeStruct((EP, 2 * od), jnp.float32),
    )(xt, xt, w1, b1, w2, b2)


def _msg_body(gs_ref, gd_ref, he_ref, wa_ref, wb_ref, wc_ref, b1_ref,
              w2_ref, b2_ref, o_ref):
    gs = gs_ref[...].astype(jnp.float32)
    gd = gd_ref[...].astype(jnp.float32)
    h = (jnp.dot(gs, wa_ref[...], preferred_element_type=jnp.float32)
         + jnp.dot(gd, wb_ref[...], preferred_element_type=jnp.float32)
         + jnp.dot(he_ref[...], wc_ref[...], preferred_element_type=jnp.float32))
    h = _silu(h + b1_ref[...])
    o_ref[...] = jnp.dot(h, w2_ref[...], preferred_element_type=jnp.float32) + b2_ref[...]


def _g_specs():
    src_spec = pl.BlockSpec((RP, P), lambda i: (i, 0))
    dst_spec = pl.BlockSpec((RP, P), lambda i: (i + E2 // R_E, 0))
    return src_spec, dst_spec


def _he_spec(off):
    return pl.BlockSpec((RP, P), lambda i: (i + off, 0))


def _msg(g2, h_edge, he_off, mp):
    src_spec, dst_spec = _g_specs()
    return pl.pallas_call(
        _msg_body,
        grid=(E2 // R_E,),
        in_specs=[src_spec, dst_spec, _he_spec(he_off)]
        + [_full(w.shape) for w in mp],
        out_specs=_rows(RP, P),
        out_shape=jax.ShapeDtypeStruct((EPH, P), jnp.float32),
    )(g2, g2, h_edge, *mp)


def _node_body(hn_ref, aga_ref, agb_ref, wa_ref, wb_ref, b1_ref, w2_ref, b2_ref,
               g_ref, be_ref, o_ref):
    hn = hn_ref[...]
    ag = aga_ref[...] + agb_ref[...]
    h = (jnp.dot(hn, wa_ref[...], preferred_element_type=jnp.float32)
         + jnp.dot(ag, wb_ref[...], preferred_element_type=jnp.float32))
    h = _silu(h + b1_ref[...])
    u = jnp.dot(h, w2_ref[...], preferred_element_type=jnp.float32) + b2_ref[...]
    x = hn + u
    m = jnp.mean(x, axis=-1, keepdims=True)
    v = jnp.mean((x - m) ** 2, axis=-1, keepdims=True)
    o_ref[...] = (x - m) * jax.lax.rsqrt(v + 1e-5) * g_ref[...] + be_ref[...]


def _node_update(h_node, agga, aggb, up):
    wa, wb, b1, w2, b2, g, be = up
    return pl.pallas_call(
        _node_body,
        grid=(N // R_N,),
        in_specs=[_rows(R_N, D), _rows(R_N, D), _rows(R_N, D),
                  _full(wa.shape), _full(wb.shape), _full(b1.shape),
                  _full(w2.shape), _full(b2.shape), _full(g.shape), _full(be.shape)],
        out_specs=_rows(R_N, D),
        out_shape=jax.ShapeDtypeStruct((N, D), jnp.float32),
    )(h_node, agga, aggb, wa, wb, b1, w2, b2, g, be)


def _edge_stage_body(gs_ref, gd_ref, he_ref, mavg_ref,
                     ea_ref, eb_ref, ec_ref, e1_ref, ew2_ref, e2_ref,
                     lg_ref, lb_ref,
                     ma_ref, mb_ref, mc_ref, m1_ref, mw2_ref, m2_ref,
                     he_out_ref, msg_out_ref):
    gs = gs_ref[...].astype(jnp.float32)
    gd = gd_ref[...].astype(jnp.float32)
    he = he_ref[...]
    h = (jnp.dot(gs, ea_ref[...], preferred_element_type=jnp.float32)
         + jnp.dot(gd, eb_ref[...], preferred_element_type=jnp.float32)
         + jnp.dot(he, ec_ref[...], preferred_element_type=jnp.float32))
    h = _silu(h + e1_ref[...])
    u = jnp.dot(h, ew2_ref[...], preferred_element_type=jnp.float32) + e2_ref[...]
    he_new = _ln_packed(he + u, lg_ref[...], lb_ref[...], mavg_ref[...])
    he_out_ref[...] = he_new
    m = (jnp.dot(gs, ma_ref[...], preferred_element_type=jnp.float32)
         + jnp.dot(gd, mb_ref[...], preferred_element_type=jnp.float32)
         + jnp.dot(he_new, mc_ref[...], preferred_element_type=jnp.float32))
    m = _silu(m + m1_ref[...])
    msg_out_ref[...] = jnp.dot(m, mw2_ref[...], preferred_element_type=jnp.float32) + m2_ref[...]


def _edge_stage(g2, h_edge, he_off, mavg, ep, mp):
    src_spec, dst_spec = _g_specs()
    ws = list(ep) + list(mp)
    return pl.pallas_call(
        _edge_stage_body,
        grid=(E2 // R_E,),
        in_specs=[src_spec, dst_spec, _he_spec(he_off), _full(mavg.shape)]
        + [_full(w.shape) for w in ws],
        out_specs=[_rows(RP, P), _rows(RP, P)],
        out_shape=[jax.ShapeDtypeStruct((EPH, P), jnp.float32),
                   jax.ShapeDtypeStruct((EPH, P), jnp.float32)],
    )(g2, g2, h_edge, mavg, *ws)


def _final_body(gs_ref, gd_ref, he_ref, mavg_ref, sel_ref,
                ea_ref, eb_ref, ec_ref, e1_ref, ew2_ref, e2_ref,
                lg_ref, lb_ref,
                m1a_ref, m1b_ref, m1c_ref, mb1_ref, m2_ref, mb2_ref, m3_ref, mb3_ref,
                r1a_ref, r1b_ref, r1c_ref, rb1_ref, r2_ref, rb2_ref, r3_ref, rb3_ref,
                merge_ref, risk_ref):
    gs = gs_ref[...].astype(jnp.float32)
    gd = gd_ref[...].astype(jnp.float32)
    he = he_ref[...]
    h = (jnp.dot(gs, ea_ref[...], preferred_element_type=jnp.float32)
         + jnp.dot(gd, eb_ref[...], preferred_element_type=jnp.float32)
         + jnp.dot(he, ec_ref[...], preferred_element_type=jnp.float32))
    h = _silu(h + e1_ref[...])
    u = jnp.dot(h, ew2_ref[...], preferred_element_type=jnp.float32) + e2_ref[...]
    he_new = _ln_packed(he + u, lg_ref[...], lb_ref[...], mavg_ref[...])

    def head(w1a, w1b, w1c, b1, w2, b2, w3, b3):
        h1 = (jnp.dot(gs, w1a, preferred_element_type=jnp.float32)
              + jnp.dot(gd, w1b, preferred_element_type=jnp.float32)
              + jnp.dot(he_new, w1c, preferred_element_type=jnp.float32))
        h1 = _silu(h1 + b1)
        h2 = _silu(jnp.dot(h1, w2, preferred_element_type=jnp.float32) + b2)
        return jnp.dot(h2 * w3, sel_ref[...],
                       preferred_element_type=jnp.float32) + b3[0, 0]

    merge_ref[...] = head(m1a_ref[...], m1b_ref[...], m1c_ref[...], mb1_ref[...],
                          m2_ref[...], mb2_ref[...], m3_ref[...], mb3_ref[...])
    risk_ref[...] = jax.nn.sigmoid(
        head(r1a_ref[...], r1b_ref[...], r1c_ref[...], rb1_ref[...],
             r2_ref[...], rb2_ref[...], r3_ref[...], rb3_ref[...]))


def _final_stage(g2, h_edge, he_off, mavg, sel, ep, hp_merge, hp_risk):
    src_spec, dst_spec = _g_specs()
    ws = list(ep) + list(hp_merge) + list(hp_risk)
    out_spec = pl.BlockSpec((RP, 2), lambda i: (i, 0))
    return pl.pallas_call(
        _final_body,
        grid=(E2 // R_E,),
        in_specs=[src_spec, dst_spec, _he_spec(he_off), _full(mavg.shape),
                  _full(sel.shape)]
        + [_full(w.shape) for w in ws],
        out_specs=[out_spec, out_spec],
        out_shape=[jax.ShapeDtypeStruct((EPH, 2), jnp.float32),
                   jax.ShapeDtypeStruct((EPH, 2), jnp.float32)],
    )(g2, g2, h_edge, mavg, sel, *ws)


_NC, _NS = 2, 16
_NW = _NC * _NS
_CH = 128
_GCHH = 6272
_CPWH = _GCHH // _NW
_GPADH = _GCHH * _CH
_KB = 8


def _make_gather_body(cpw):
    def _gather_body(table, idx3, out, idx_v, *rest):
        bufs = rest[:_KB]
        gsem, wsem = rest[_KB], rest[_KB + 1]
        w = lax.axis_index("s") * _NC + lax.axis_index("c")
        pltpu.sync_copy(idx3.at[w], idx_v)
        base = w * cpw
        ngrp = (cpw + _KB - 1) // _KB

        def grp(g, carry):
            for b in range(_KB):
                j = g * _KB + b

                @pl.when(j < cpw)
                def _():
                    @pl.when(g > 0)
                    def _():
                        pltpu.make_async_copy(
                            bufs[b], out.at[pl.ds((base + j - _KB) * _CH, _CH)],
                            wsem.at[b]).wait()
                    pltpu.async_copy(table.at[idx_v.at[j]], bufs[b], gsem.at[b])
            for b in range(_KB):
                j = g * _KB + b

                @pl.when(j < cpw)
                def _():
                    pltpu.make_async_copy(table.at[idx_v.at[j]], bufs[b],
                                          gsem.at[b]).wait()
                    pltpu.async_copy(bufs[b],
                                     out.at[pl.ds((base + j) * _CH, _CH)],
                                     wsem.at[b])
            return carry

        lax.fori_loop(0, ngrp, grp, 0)
        ngrp = (cpw + _KB - 1) // _KB
        ntail = cpw - (ngrp - 1) * _KB
        for b in range(_KB):
            j = (ngrp - 1) * _KB + b if b < ntail else (ngrp - 2) * _KB + b
            pltpu.make_async_copy(bufs[b], out.at[pl.ds((base + j) * _CH, _CH)],
                                  wsem.at[b]).wait()

    return _gather_body


def _gather(h_node, idx3, cpw, gpad):
    mesh = plsc.VectorSubcoreMesh(core_axis_name="c", subcore_axis_name="s")
    return pl.kernel(
        _make_gather_body(cpw),
        mesh=mesh,
        compiler_params=pltpu.CompilerParams(use_tc_tiling_on_sc=False),
        out_type=jax.ShapeDtypeStruct((gpad, D), jnp.float32),
        scratch_types=(
            [pltpu.VMEM((cpw, _CH), jnp.int32)]
            + [pltpu.VMEM((_CH, D), jnp.float32) for _ in range(_KB)]
            + [pltpu.SemaphoreType.DMA((_KB,)), pltpu.SemaphoreType.DMA((_KB,))]
        ),
    )(h_node, idx3)


_HQ = D // 4
_SCH = E // _CH
_SCHH = E2 // _CH
_SCHPH = 3136
_CPTH = _SCHPH // _NS
_AROW = 51200
_ZB = _AROW // _NS // _CH
_KS = 8
_HCH = _CH // 2


def _make_scatter_body(nsch, cpt):
  def _scatter_body(msg2, dst2, agg, idx_v, zbuf, acc, *rest):
      vbufs = rest[:_KS]
      lsem, ssem = rest[_KS], rest[_KS + 1]
      c = lax.axis_index("c")
      s = lax.axis_index("s")
      base = s * cpt
      pltpu.sync_copy(dst2.at[pl.ds(base, cpt)], idx_v)

      def zb(i, carry):
          zbuf[i, pl.ds(0, 16)] = jnp.zeros((16,), jnp.float32)
          return carry

      lax.fori_loop(0, _CH, zb, 0)

      ngrp = (cpt + _KS - 1) // _KS

      for half in range(2):
          col0 = c * _HQ + half * 2 * _HQ

          def zg(k, carry):
              pltpu.sync_copy(zbuf, acc.at[pl.ds((s * _ZB + k) * _CH, _CH)])
              return carry

          lax.fori_loop(0, _ZB, zg, 0)
          plsc.subcore_barrier()

          def ld(j, b):
              r0 = (base + j) * _HCH
              even = pltpu.async_copy(
                  msg2.at[pl.ds(r0, _HCH), pl.ds(col0, _HQ)],
                  vbufs[b].at[pl.ds(0, _HCH)], lsem.at[b])
              odd = pltpu.async_copy(
                  msg2.at[pl.ds(r0, _HCH), pl.ds(D + col0, _HQ)],
                  vbufs[b].at[pl.ds(_HCH, _HCH)], lsem.at[b])
              return even, odd

          def grp(g, carry):
              for b in range(_KS):
                  j = g * _KS + b

                  @pl.when((j < cpt) & (base + j < nsch))
                  def _():
                      @pl.when(g > 0)
                      def _():
                          pltpu.make_async_copy(vbufs[b], acc.at[pl.ds(0, _CH)],
                                                ssem.at[b]).wait()
                      ld(j, b)
              for b in range(_KS):
                  j = g * _KS + b

                  @pl.when((j < cpt) & (base + j < nsch))
                  def _():
                      r0 = (base + j) * _HCH
                      pltpu.make_async_copy(
                          msg2.at[pl.ds(r0, _HCH), pl.ds(col0, _HQ)],
                          vbufs[b].at[pl.ds(0, _HCH)], lsem.at[b]).wait()
                      pltpu.make_async_copy(
                          msg2.at[pl.ds(r0, _HCH), pl.ds(D + col0, _HQ)],
                          vbufs[b].at[pl.ds(_HCH, _HCH)], lsem.at[b]).wait()
                      pltpu.async_copy(vbufs[b], acc.at[idx_v.at[j]], ssem.at[b],
                                       add=True)
              return carry

          lax.fori_loop(0, ngrp, grp, 0)
          for b in range(_KS):
              last = (nsch - 1 - base - b) // _KS

              @pl.when(last >= 0)
              def _():
                  pltpu.make_async_copy(vbufs[b], acc.at[pl.ds(0, _CH)],
                                        ssem.at[b]).wait()
          plsc.subcore_barrier()
          nr = N // _NS
          pltpu.sync_copy(acc.at[pl.ds(s * nr, nr)],
                          agg.at[pl.ds(s * nr, nr), pl.ds(col0, _HQ)])
          plsc.subcore_barrier()

  return _scatter_body


def _scatter_add(msg2, dst2, nsch, cpt):
    mesh = plsc.VectorSubcoreMesh(core_axis_name="c", subcore_axis_name="s")
    return pl.kernel(
        _make_scatter_body(nsch, cpt),
        mesh=mesh,
        compiler_params=pltpu.CompilerParams(use_tc_tiling_on_sc=False),
        out_type=jax.ShapeDtypeStruct((N, D), jnp.float32),
        scratch_types=(
            [pltpu.VMEM((cpt, _CH), jnp.int32),
             pltpu.VMEM((_CH, _HQ), jnp.float32),
             pltpu.VMEM_SHARED((_AROW, _HQ), jnp.float32)]
            + [pltpu.VMEM((_CH, _HQ), jnp.float32) for _ in range(_KS)]
            + [pltpu.SemaphoreType.DMA((_KS,)), pltpu.SemaphoreType.DMA((_KS,))]
        ),
    )(msg2, dst2)


def _bd(w):
    return jnp.kron(jnp.eye(2, dtype=jnp.float32), w)


def _t2(b):
    return jnp.tile(b.reshape(1, -1), (1, 2))


def _split3(w):
    return w[:D], w[D:2 * D], w[2 * D:]


def _prep_node_embed(ps):
    (w1, b1), (w2, b2) = ps
    return w1, b1.reshape(1, -1), w2, b2.reshape(1, -1)


def _prep_edge_embed(ps):
    (w1, b1), (w2, b2) = ps
    return w1, b1.reshape(1, -1), w2, b2.reshape(1, -1)


def _prep_msg(ps):
    (w1, b1), (w2, b2) = ps
    wa, wb, wc = _split3(w1)
    return _bd(wa), _bd(wb), _bd(wc), _t2(b1), _bd(w2), _t2(b2)


def _prep_upd(ps, norm):
    (w1, b1), (w2, b2) = ps
    wa, wb = w1[:D], w1[D:]
    g, be = norm
    return wa, wb, b1.reshape(1, -1), w2, b2.reshape(1, -1), g.reshape(1, -1), be.reshape(1, -1)


def _prep_edge(ps, norm):
    (w1, b1), (w2, b2) = ps
    wa, wb, wc = _split3(w1)
    g, be = norm
    return _bd(wa), _bd(wb), _bd(wc), _t2(b1), _bd(w2), _t2(b2), _t2(g), _t2(be)


def _prep_head(ps):
    (w1, b1), (w2, b2), (w3, b3) = ps
    wa, wb, wc = _split3(w1)
    return (_bd(wa), _bd(wb), _bd(wc), _t2(b1), _bd(w2), _t2(b2),
            _t2(w3.reshape(1, -1)), b3.reshape(1, 1))


def kernel(node_feat, edge_index, edge_feat, params):
    src = edge_index[:, 0]
    dst = edge_index[:, 1]
    def _halves(a):
        return jnp.concatenate([a[:EP].reshape(_SCH, _HCH),
                                a[EP:].reshape(_SCH, _HCH)], axis=1)

    src_p = jnp.stack([src[:EP], src[EP:]], axis=1).reshape(E)
    dst_p = jnp.stack([dst[:EP], dst[EP:]], axis=1).reshape(E)
    zg = jnp.zeros((_GPADH - 2 * E2,), jnp.int32)
    idx3a = jnp.concatenate(
        [src_p[:E2], dst_p[:E2], zg]).reshape(_NW, _CPWH, _CH)
    idx3b = jnp.concatenate(
        [src_p[E2:], dst_p[E2:], zg]).reshape(_NW, _CPWH, _CH)
    d2 = _halves(dst)
    zs = jnp.zeros((_SCHPH - _SCHH, _CH), jnp.int32)
    dst2a = jnp.concatenate([d2[:_SCHH], zs])
    dst2b = jnp.concatenate([d2[_SCHH:], zs])

    mavg = _bd(jnp.full((D, D), 1.0 / D, jnp.float32))
    sel = _bd(jnp.ones((D, 1), jnp.float32))

    ne = _prep_node_embed(params["node_embed"])
    ee = _prep_edge_embed(params["edge_embed"])
    layers = [{
        "msg": _prep_msg(lp["msg"]),
        "upd": _prep_upd(lp["upd"], lp["node_norm"]),
        "edge": _prep_edge(lp["edge_upd"], lp["edge_norm"]),
    } for lp in params["layers"]]
    hp_merge = _prep_head(params["merge_head"])
    hp_risk = _prep_head(params["risk_head"])

    h_node = _mlp2(node_feat, *ne, R_N)
    h_edge = _mlp2_pack(jnp.swapaxes(edge_feat, 0, 1), *ee, CB_EMB)

    def gath(hn, i3):
        return _gather(hn, i3, _CPWH, _GPADH).reshape(_GPADH // 2, P)

    off_b = EPH // RP
    ga = gath(h_node, idx3a)
    gb = gath(h_node, idx3b)
    msga = _msg(ga, h_edge, 0, layers[0]["msg"])
    msgb = _msg(gb, h_edge, off_b, layers[0]["msg"])
    hea, oa = h_edge, 0
    heb, ob = h_edge, off_b
    for i in range(6):
        agga = _scatter_add(msga, dst2a, _SCHH, _CPTH)
        aggb = _scatter_add(msgb, dst2b, _SCHH, _CPTH)
        h_node = _node_update(h_node, agga, aggb, layers[i]["upd"])
        ga = gath(h_node, idx3a)
        gb = gath(h_node, idx3b)
        if i < 5:
            hea, msga = _edge_stage(ga, hea, oa, mavg,
                                    layers[i]["edge"], layers[i + 1]["msg"])
            heb, msgb = _edge_stage(gb, heb, ob, mavg,
                                    layers[i]["edge"], layers[i + 1]["msg"])
            oa = ob = 0
        else:
            ma, ra = _final_stage(ga, hea, oa, mavg, sel,
                                  layers[i]["edge"], hp_merge, hp_risk)
            mb, rb = _final_stage(gb, heb, ob, mavg, sel,
                                  layers[i]["edge"], hp_merge, hp_risk)
    merge = jnp.concatenate([ma[:, 0], mb[:, 0], ma[:, 1], mb[:, 1]])
    risk = jnp.concatenate([ra[:, 0], rb[:, 0], ra[:, 1], rb[:, 1]])
    return (merge, risk)


# speedup vs baseline: 1.5239x; 1.0320x over previous
import jax
import jax.numpy as jnp
from jax import lax
from jax.experimental import pallas as pl
from jax.experimental.pallas import tpu as pltpu
from jax.experimental.pallas import tpu_sc as plsc

N = 50000
E = 800000
D = 64
P = 2 * D

R_E = 8000
R_N = 2000
RP = R_E // 2
EP = E // 2
E2 = E // 2
EPH = E2 // 2
CB_EMB = 16000


def _silu(x):
    return x * jax.nn.sigmoid(x)


def _ln_packed(x, g, b, mavg):
    m = jnp.dot(x, mavg, preferred_element_type=jnp.float32)
    d = x - m
    v = jnp.dot(d * d, mavg, preferred_element_type=jnp.float32)
    return d * jax.lax.rsqrt(v + 1e-5) * g + b


def _full(shape):
    return pl.BlockSpec(shape, lambda i: tuple(0 for _ in shape))


def _rows(r, w):
    return pl.BlockSpec((r, w), lambda i: (i, 0))


def _mlp2_body(x_ref, w1_ref, b1_ref, w2_ref, b2_ref, o_ref):
    h = jnp.dot(x_ref[...], w1_ref[...], preferred_element_type=jnp.float32)
    h = _silu(h + b1_ref[...])
    o_ref[...] = jnp.dot(h, w2_ref[...], preferred_element_type=jnp.float32) + b2_ref[...]


def _mlp2(x, w1, b1, w2, b2, r):
    n = x.shape[0]
    od = w2.shape[1]
    return pl.pallas_call(
        _mlp2_body,
        grid=(n // r,),
        in_specs=[
            _rows(r, x.shape[1]),
            _full(w1.shape), _full(b1.shape), _full(w2.shape), _full(b2.shape),
        ],
        out_specs=_rows(r, od),
        out_shape=jax.ShapeDtypeStruct((n, od), jnp.float32),
    )(x, w1, b1, w2, b2)


def _mlp2_pack_body(xa_ref, xb_ref, w1_ref, b1_ref, w2_ref, b2_ref, o_ref):
    def m(xt):
        h = lax.dot_general(xt, w1_ref[...], (((0,), (0,)), ((), ())),
                            preferred_element_type=jnp.float32)
        h = _silu(h + b1_ref[...])
        return jnp.dot(h, w2_ref[...], preferred_element_type=jnp.float32) + b2_ref[...]

    o_ref[...] = jnp.concatenate([m(xa_ref[...]), m(xb_ref[...])], axis=1)


def _mlp2_pack(xt, w1, b1, w2, b2, cb):
    nf = xt.shape[0]
    od = w2.shape[1]
    ng = EP // cb
    a_spec = pl.BlockSpec((nf, cb), lambda i: (0, i))
    b_spec = pl.BlockSpec((nf, cb), lambda i: (0, i + ng))
    return pl.pallas_call(
        _mlp2_pack_body,
        grid=(ng,),
        in_specs=[a_spec, b_spec,
                  _full(w1.shape), _full(b1.shape), _full(w2.shape), _full(b2.shape)],
        out_specs=pl.BlockSpec((cb, 2 * od), lambda i: (i, 0)),
        out_shape=jax.ShapeDtypeStruct((EP, 2 * od), jnp.float32),
    )(xt, xt, w1, b1, w2, b2)


def _msg_body(gs_ref, gd_ref, he_ref, wa_ref, wb_ref, wc_ref, b1_ref,
              w2_ref, b2_ref, o_ref):
    gs = gs_ref[...].astype(jnp.float32)
    gd = gd_ref[...].astype(jnp.float32)
    h = (jnp.dot(gs, wa_ref[...], preferred_element_type=jnp.float32)
         + jnp.dot(gd, wb_ref[...], preferred_element_type=jnp.float32)
         + jnp.dot(he_ref[...], wc_ref[...], preferred_element_type=jnp.float32))
    h = _silu(h + b1_ref[...])
    o_ref[...] = jnp.dot(h, w2_ref[...], preferred_element_type=jnp.float32) + b2_ref[...]


def _g_specs():
    src_spec = pl.BlockSpec((RP, P), lambda i: (i, 0))
    dst_spec = pl.BlockSpec((RP, P), lambda i: (i + E2 // R_E, 0))
    return src_spec, dst_spec


def _he_spec(off):
    return pl.BlockSpec((RP, P), lambda i: (i + off, 0))


def _msg(g2, h_edge, he_off, mp):
    src_spec, dst_spec = _g_specs()
    return pl.pallas_call(
        _msg_body,
        grid=(E2 // R_E,),
        in_specs=[src_spec, dst_spec, _he_spec(he_off)]
        + [_full(w.shape) for w in mp],
        out_specs=_rows(RP, P),
        out_shape=jax.ShapeDtypeStruct((EPH, P), jnp.float32),
    )(g2, g2, h_edge, *mp)


def _node_body(hn_ref, aga_ref, agb_ref, wa_ref, wb_ref, b1_ref, w2_ref, b2_ref,
               g_ref, be_ref, o_ref):
    hn = hn_ref[...]
    ag = aga_ref[...] + agb_ref[...]
    h = (jnp.dot(hn, wa_ref[...], preferred_element_type=jnp.float32)
         + jnp.dot(ag, wb_ref[...], preferred_element_type=jnp.float32))
    h = _silu(h + b1_ref[...])
    u = jnp.dot(h, w2_ref[...], preferred_element_type=jnp.float32) + b2_ref[...]
    x = hn + u
    m = jnp.mean(x, axis=-1, keepdims=True)
    v = jnp.mean((x - m) ** 2, axis=-1, keepdims=True)
    o_ref[...] = (x - m) * jax.lax.rsqrt(v + 1e-5) * g_ref[...] + be_ref[...]


def _node_update(h_node, agga, aggb, up):
    wa, wb, b1, w2, b2, g, be = up
    return pl.pallas_call(
        _node_body,
        grid=(N // R_N,),
        in_specs=[_rows(R_N, D), _rows(R_N, D), _rows(R_N, D),
                  _full(wa.shape), _full(wb.shape), _full(b1.shape),
                  _full(w2.shape), _full(b2.shape), _full(g.shape), _full(be.shape)],
        out_specs=_rows(R_N, D),
        out_shape=jax.ShapeDtypeStruct((N, D), jnp.float32),
    )(h_node, agga, aggb, wa, wb, b1, w2, b2, g, be)


def _edge_stage_body(gs_ref, gd_ref, he_ref, mavg_ref,
                     ea_ref, eb_ref, ec_ref, e1_ref, ew2_ref, e2_ref,
                     lg_ref, lb_ref,
                     ma_ref, mb_ref, mc_ref, m1_ref, mw2_ref, m2_ref,
                     he_out_ref, msg_out_ref):
    gs = gs_ref[...].astype(jnp.float32)
    gd = gd_ref[...].astype(jnp.float32)
    he = he_ref[...]
    h = (jnp.dot(gs, ea_ref[...], preferred_element_type=jnp.float32)
         + jnp.dot(gd, eb_ref[...], preferred_element_type=jnp.float32)
         + jnp.dot(he, ec_ref[...], preferred_element_type=jnp.float32))
    h = _silu(h + e1_ref[...])
    u = jnp.dot(h, ew2_ref[...], preferred_element_type=jnp.float32) + e2_ref[...]
    he_new = _ln_packed(he + u, lg_ref[...], lb_ref[...], mavg_ref[...])
    he_out_ref[...] = he_new
    m = (jnp.dot(gs, ma_ref[...], preferred_element_type=jnp.float32)
         + jnp.dot(gd, mb_ref[...], preferred_element_type=jnp.float32)
         + jnp.dot(he_new, mc_ref[...], preferred_element_type=jnp.float32))
    m = _silu(m + m1_ref[...])
    msg_out_ref[...] = jnp.dot(m, mw2_ref[...], preferred_element_type=jnp.float32) + m2_ref[...]


def _edge_stage(g2, h_edge, he_off, mavg, ep, mp):
    src_spec, dst_spec = _g_specs()
    ws = list(ep) + list(mp)
    return pl.pallas_call(
        _edge_stage_body,
        grid=(E2 // R_E,),
        in_specs=[src_spec, dst_spec, _he_spec(he_off), _full(mavg.shape)]
        + [_full(w.shape) for w in ws],
        out_specs=[_rows(RP, P), _rows(RP, P)],
        out_shape=[jax.ShapeDtypeStruct((EPH, P), jnp.float32),
                   jax.ShapeDtypeStruct((EPH, P), jnp.float32)],
    )(g2, g2, h_edge, mavg, *ws)


def _final_body(gs_ref, gd_ref, he_ref, mavg_ref, sel_ref,
                ea_ref, eb_ref, ec_ref, e1_ref, ew2_ref, e2_ref,
                lg_ref, lb_ref,
                m1a_ref, m1b_ref, m1c_ref, mb1_ref, m2_ref, mb2_ref, m3_ref, mb3_ref,
                r1a_ref, r1b_ref, r1c_ref, rb1_ref, r2_ref, rb2_ref, r3_ref, rb3_ref,
                merge_ref, risk_ref):
    gs = gs_ref[...].astype(jnp.float32)
    gd = gd_ref[...].astype(jnp.float32)
    he = he_ref[...]
    h = (jnp.dot(gs, ea_ref[...], preferred_element_type=jnp.float32)
         + jnp.dot(gd, eb_ref[...], preferred_element_type=jnp.float32)
         + jnp.dot(he, ec_ref[...], preferred_element_type=jnp.float32))
    h = _silu(h + e1_ref[...])
    u = jnp.dot(h, ew2_ref[...], preferred_element_type=jnp.float32) + e2_ref[...]
    he_new = _ln_packed(he + u, lg_ref[...], lb_ref[...], mavg_ref[...])

    def head(w1a, w1b, w1c, b1, w2, b2, w3, b3):
        h1 = (jnp.dot(gs, w1a, preferred_element_type=jnp.float32)
              + jnp.dot(gd, w1b, preferred_element_type=jnp.float32)
              + jnp.dot(he_new, w1c, preferred_element_type=jnp.float32))
        h1 = _silu(h1 + b1)
        h2 = _silu(jnp.dot(h1, w2, preferred_element_type=jnp.float32) + b2)
        return jnp.dot(h2 * w3, sel_ref[...],
                       preferred_element_type=jnp.float32) + b3[0, 0]

    merge_ref[...] = head(m1a_ref[...], m1b_ref[...], m1c_ref[...], mb1_ref[...],
                          m2_ref[...], mb2_ref[...], m3_ref[...], mb3_ref[...])
    risk_ref[...] = jax.nn.sigmoid(
        head(r1a_ref[...], r1b_ref[...], r1c_ref[...], rb1_ref[...],
             r2_ref[...], rb2_ref[...], r3_ref[...], rb3_ref[...]))


def _final_stage(g2, h_edge, he_off, mavg, sel, ep, hp_merge, hp_risk):
    src_spec, dst_spec = _g_specs()
    ws = list(ep) + list(hp_merge) + list(hp_risk)
    out_spec = pl.BlockSpec((RP, 2), lambda i: (i, 0))
    return pl.pallas_call(
        _final_body,
        grid=(E2 // R_E,),
        in_specs=[src_spec, dst_spec, _he_spec(he_off), _full(mavg.shape),
                  _full(sel.shape)]
        + [_full(w.shape) for w in ws],
        out_specs=[out_spec, out_spec],
        out_shape=[jax.ShapeDtypeStruct((EPH, 2), jnp.float32),
                   jax.ShapeDtypeStruct((EPH, 2), jnp.float32)],
    )(g2, g2, h_edge, mavg, sel, *ws)


_NC, _NS = 2, 16
_NW = _NC * _NS
_CH = 128
_GCHH = 6272
_CPWH = _GCHH // _NW
_GPADH = _GCHH * _CH
_KB = 8


def _make_gather_body(cpw, choff):
    def _gather_body(table, idxc, out, idx_v, *rest):
        bufs = rest[:_KB]
        gsem, wsem = rest[_KB], rest[_KB + 1]
        w = lax.axis_index("s") * _NC + lax.axis_index("c")
        pltpu.sync_copy(idxc.at[pl.ds(choff + w * cpw, cpw)], idx_v)
        base = w * cpw
        ngrp = (cpw + _KB - 1) // _KB

        def grp(g, carry):
            for b in range(_KB):
                j = g * _KB + b

                @pl.when(j < cpw)
                def _():
                    @pl.when(g > 0)
                    def _():
                        pltpu.make_async_copy(
                            bufs[b], out.at[pl.ds((base + j - _KB) * _CH, _CH)],
                            wsem.at[b]).wait()
                    pltpu.async_copy(table.at[idx_v.at[j]], bufs[b], gsem.at[b])
            for b in range(_KB):
                j = g * _KB + b

                @pl.when(j < cpw)
                def _():
                    pltpu.make_async_copy(table.at[idx_v.at[j]], bufs[b],
                                          gsem.at[b]).wait()
                    pltpu.async_copy(bufs[b],
                                     out.at[pl.ds((base + j) * _CH, _CH)],
                                     wsem.at[b])
            return carry

        lax.fori_loop(0, ngrp, grp, 0)
        ngrp = (cpw + _KB - 1) // _KB
        ntail = cpw - (ngrp - 1) * _KB
        for b in range(_KB):
            j = (ngrp - 1) * _KB + b if b < ntail else (ngrp - 2) * _KB + b
            pltpu.make_async_copy(bufs[b], out.at[pl.ds((base + j) * _CH, _CH)],
                                  wsem.at[b]).wait()

    return _gather_body


def _gather(h_node, idxc, cpw, gpad, choff):
    mesh = plsc.VectorSubcoreMesh(core_axis_name="c", subcore_axis_name="s")
    return pl.kernel(
        _make_gather_body(cpw, choff),
        mesh=mesh,
        compiler_params=pltpu.CompilerParams(use_tc_tiling_on_sc=False),
        out_type=jax.ShapeDtypeStruct((gpad, D), jnp.float32),
        scratch_types=(
            [pltpu.VMEM((cpw, _CH), jnp.int32)]
            + [pltpu.VMEM((_CH, D), jnp.float32) for _ in range(_KB)]
            + [pltpu.SemaphoreType.DMA((_KB,)), pltpu.SemaphoreType.DMA((_KB,))]
        ),
    )(h_node, idxc)


_HQ = D // 4
_SCH = E // _CH
_SCHH = E2 // _CH
_SCHPH = 3136
_CPTH = _SCHPH // _NS
_AROW = 51200
_ZB = _AROW // _NS // _CH
_KS = 8
_HCH = _CH // 2


def _make_scatter_body(nsch, cpt, choff):
  def _scatter_body(msg2, dst2, agg, idx_v, zbuf, acc, *rest):
      vbufs = rest[:_KS]
      lsem, ssem = rest[_KS], rest[_KS + 1]
      c = lax.axis_index("c")
      s = lax.axis_index("s")
      base = s * cpt
      pltpu.sync_copy(dst2.at[pl.ds(choff + base, cpt)], idx_v)

      def zb(i, carry):
          zbuf[i, pl.ds(0, 16)] = jnp.zeros((16,), jnp.float32)
          return carry

      lax.fori_loop(0, _CH, zb, 0)

      ngrp = (cpt + _KS - 1) // _KS

      for half in range(2):
          col0 = c * _HQ + half * 2 * _HQ

          def zg(k, carry):
              pltpu.sync_copy(zbuf, acc.at[pl.ds((s * _ZB + k) * _CH, _CH)])
              return carry

          lax.fori_loop(0, _ZB, zg, 0)
          plsc.subcore_barrier()

          def ld(j, b):
              r0 = (base + j) * _HCH
              even = pltpu.async_copy(
                  msg2.at[pl.ds(r0, _HCH), pl.ds(col0, _HQ)],
                  vbufs[b].at[pl.ds(0, _HCH)], lsem.at[b])
              odd = pltpu.async_copy(
                  msg2.at[pl.ds(r0, _HCH), pl.ds(D + col0, _HQ)],
                  vbufs[b].at[pl.ds(_HCH, _HCH)], lsem.at[b])
              return even, odd

          def grp(g, carry):
              for b in range(_KS):
                  j = g * _KS + b

                  @pl.when((j < cpt) & (base + j < nsch))
                  def _():
                      @pl.when(g > 0)
                      def _():
                          pltpu.make_async_copy(vbufs[b], acc.at[pl.ds(0, _CH)],
                                                ssem.at[b]).wait()
                      ld(j, b)
              for b in range(_KS):
                  j = g * _KS + b

                  @pl.when((j < cpt) & (base + j < nsch))
                  def _():
                      r0 = (base + j) * _HCH
                      pltpu.make_async_copy(
                          msg2.at[pl.ds(r0, _HCH), pl.ds(col0, _HQ)],
                          vbufs[b].at[pl.ds(0, _HCH)], lsem.at[b]).wait()
                      pltpu.make_async_copy(
                          msg2.at[pl.ds(r0, _HCH), pl.ds(D + col0, _HQ)],
                          vbufs[b].at[pl.ds(_HCH, _HCH)], lsem.at[b]).wait()
                      pltpu.async_copy(vbufs[b], acc.at[idx_v.at[j]], ssem.at[b],
                                       add=True)
              return carry

          lax.fori_loop(0, ngrp, grp, 0)
          for b in range(_KS):
              last = (nsch - 1 - base - b) // _KS

              @pl.when(last >= 0)
              def _():
                  pltpu.make_async_copy(vbufs[b], acc.at[pl.ds(0, _CH)],
                                        ssem.at[b]).wait()
          plsc.subcore_barrier()
          nr = N // _NS
          pltpu.sync_copy(acc.at[pl.ds(s * nr, nr)],
                          agg.at[pl.ds(s * nr, nr), pl.ds(col0, _HQ)])
          plsc.subcore_barrier()

  return _scatter_body


def _scatter_add(msg2, dst2, nsch, cpt, choff):
    mesh = plsc.VectorSubcoreMesh(core_axis_name="c", subcore_axis_name="s")
    return pl.kernel(
        _make_scatter_body(nsch, cpt, choff),
        mesh=mesh,
        compiler_params=pltpu.CompilerParams(use_tc_tiling_on_sc=False),
        out_type=jax.ShapeDtypeStruct((N, D), jnp.float32),
        scratch_types=(
            [pltpu.VMEM((cpt, _CH), jnp.int32),
             pltpu.VMEM((_CH, _HQ), jnp.float32),
             pltpu.VMEM_SHARED((_AROW, _HQ), jnp.float32)]
            + [pltpu.VMEM((_CH, _HQ), jnp.float32) for _ in range(_KS)]
            + [pltpu.SemaphoreType.DMA((_KS,)), pltpu.SemaphoreType.DMA((_KS,))]
        ),
    )(msg2, dst2)


def _bd(w):
    return jnp.kron(jnp.eye(2, dtype=jnp.float32), w)


def _t2(b):
    return jnp.tile(b.reshape(1, -1), (1, 2))


def _split3(w):
    return w[:D], w[D:2 * D], w[2 * D:]


def _prep_node_embed(ps):
    (w1, b1), (w2, b2) = ps
    return w1, b1.reshape(1, -1), w2, b2.reshape(1, -1)


def _prep_edge_embed(ps):
    (w1, b1), (w2, b2) = ps
    return w1, b1.reshape(1, -1), w2, b2.reshape(1, -1)


def _prep_msg(ps):
    (w1, b1), (w2, b2) = ps
    wa, wb, wc = _split3(w1)
    return _bd(wa), _bd(wb), _bd(wc), _t2(b1), _bd(w2), _t2(b2)


def _prep_upd(ps, norm):
    (w1, b1), (w2, b2) = ps
    wa, wb = w1[:D], w1[D:]
    g, be = norm
    return wa, wb, b1.reshape(1, -1), w2, b2.reshape(1, -1), g.reshape(1, -1), be.reshape(1, -1)


def _prep_edge(ps, norm):
    (w1, b1), (w2, b2) = ps
    wa, wb, wc = _split3(w1)
    g, be = norm
    return _bd(wa), _bd(wb), _bd(wc), _t2(b1), _bd(w2), _t2(b2), _t2(g), _t2(be)


def _prep_head(ps):
    (w1, b1), (w2, b2), (w3, b3) = ps
    wa, wb, wc = _split3(w1)
    return (_bd(wa), _bd(wb), _bd(wc), _t2(b1), _bd(w2), _t2(b2),
            _t2(w3.reshape(1, -1)), b3.reshape(1, 1))


def kernel(node_feat, edge_index, edge_feat, params):
    src = edge_index[:, 0]
    dst = edge_index[:, 1]
    def _halves(a):
        return jnp.concatenate([a[:EP].reshape(_SCH, _HCH),
                                a[EP:].reshape(_SCH, _HCH)], axis=1)

    e4 = E2 // 2

    def _ilv(a, lo):
        return jnp.stack([a[lo:lo + e4], a[EP + lo:EP + lo + e4]],
                         axis=1).reshape(_SCHH, _CH)

    zg = jnp.zeros((_GCHH - 2 * _SCHH, _CH), jnp.int32)
    idxc = jnp.concatenate([_ilv(src, 0), _ilv(dst, 0), zg,
                            _ilv(src, e4), _ilv(dst, e4), zg])
    d2 = _halves(dst)
    zs = jnp.zeros((_SCHPH - _SCHH, _CH), jnp.int32)
    dst2 = jnp.concatenate([d2[:_SCHH], zs, d2[_SCHH:], zs])

    mavg = _bd(jnp.full((D, D), 1.0 / D, jnp.float32))
    sel = _bd(jnp.ones((D, 1), jnp.float32))

    ne = _prep_node_embed(params["node_embed"])
    ee = _prep_edge_embed(params["edge_embed"])
    layers = [{
        "msg": _prep_msg(lp["msg"]),
        "upd": _prep_upd(lp["upd"], lp["node_norm"]),
        "edge": _prep_edge(lp["edge_upd"], lp["edge_norm"]),
    } for lp in params["layers"]]
    hp_merge = _prep_head(params["merge_head"])
    hp_risk = _prep_head(params["risk_head"])

    h_node = _mlp2(node_feat, *ne, R_N)
    h_edge = _mlp2_pack(jnp.swapaxes(edge_feat, 0, 1), *ee, CB_EMB)

    def gath(hn, choff):
        return _gather(hn, idxc, _CPWH, _GPADH, choff).reshape(_GPADH // 2, P)

    off_b = EPH // RP
    ga = gath(h_node, 0)
    gb = gath(h_node, _GCHH)
    msga = _msg(ga, h_edge, 0, layers[0]["msg"])
    msgb = _msg(gb, h_edge, off_b, layers[0]["msg"])
    hea, oa = h_edge, 0
    heb, ob = h_edge, off_b
    for i in range(6):
        agga = _scatter_add(msga, dst2, _SCHH, _CPTH, 0)
        aggb = _scatter_add(msgb, dst2, _SCHH, _CPTH, _SCHPH)
        h_node = _node_update(h_node, agga, aggb, layers[i]["upd"])
        ga = gath(h_node, 0)
        gb = gath(h_node, _GCHH)
        if i < 5:
            hea, msga = _edge_stage(ga, hea, oa, mavg,
                                    layers[i]["edge"], layers[i + 1]["msg"])
            heb, msgb = _edge_stage(gb, heb, ob, mavg,
                                    layers[i]["edge"], layers[i + 1]["msg"])
            oa = ob = 0
        else:
            ma, ra = _final_stage(ga, hea, oa, mavg, sel,
                                  layers[i]["edge"], hp_merge, hp_risk)
            mb, rb = _final_stage(gb, heb, ob, mavg, sel,
                                  layers[i]["edge"], hp_merge, hp_risk)
    merge = jnp.concatenate([ma[:, 0], mb[:, 0], ma[:, 1], mb[:, 1]])
    risk = jnp.concatenate([ra[:, 0], rb[:, 0], ra[:, 1], rb[:, 1]])
    return (merge, risk)


# speedup vs baseline: 1.6063x; 1.0541x over previous
import jax
import jax.numpy as jnp
from jax import lax
from jax.experimental import pallas as pl
from jax.experimental.pallas import tpu as pltpu
from jax.experimental.pallas import tpu_sc as plsc

N = 50000
E = 800000
D = 64
P = 2 * D

R_E = 16000
R_N = 2000
RP = R_E // 2
EP = E // 2
CB_EMB = 16000


def _silu(x):
    return x * jax.nn.sigmoid(x)


def _ln_packed(x, g, b, mavg):
    m = jnp.dot(x, mavg, preferred_element_type=jnp.float32)
    d = x - m
    v = jnp.dot(d * d, mavg, preferred_element_type=jnp.float32)
    return d * jax.lax.rsqrt(v + 1e-5) * g + b


def _full(shape):
    return pl.BlockSpec(shape, lambda i: tuple(0 for _ in shape))


def _rows(r, w):
    return pl.BlockSpec((r, w), lambda i: (i, 0))


def _mlp2_body(x_ref, w1_ref, b1_ref, w2_ref, b2_ref, o_ref):
    h = jnp.dot(x_ref[...], w1_ref[...], preferred_element_type=jnp.float32)
    h = _silu(h + b1_ref[...])
    o_ref[...] = jnp.dot(h, w2_ref[...], preferred_element_type=jnp.float32) + b2_ref[...]


def _mlp2(x, w1, b1, w2, b2, r):
    n = x.shape[0]
    od = w2.shape[1]
    return pl.pallas_call(
        _mlp2_body,
        grid=(n // r,),
        in_specs=[
            _rows(r, x.shape[1]),
            _full(w1.shape), _full(b1.shape), _full(w2.shape), _full(b2.shape),
        ],
        out_specs=_rows(r, od),
        out_shape=jax.ShapeDtypeStruct((n, od), jnp.float32),
    )(x, w1, b1, w2, b2)


def _mlp2_pack_body(xa_ref, xb_ref, w1_ref, b1_ref, w2_ref, b2_ref, o_ref):
    def m(xt):
        h = lax.dot_general(xt, w1_ref[...], (((0,), (0,)), ((), ())),
                            preferred_element_type=jnp.float32)
        h = _silu(h + b1_ref[...])
        return jnp.dot(h, w2_ref[...], preferred_element_type=jnp.float32) + b2_ref[...]

    o_ref[...] = jnp.concatenate([m(xa_ref[...]), m(xb_ref[...])], axis=1)


def _mlp2_pack(xt, w1, b1, w2, b2, cb):
    nf = xt.shape[0]
    od = w2.shape[1]
    ng = EP // cb
    a_spec = pl.BlockSpec((nf, cb), lambda i: (0, i))
    b_spec = pl.BlockSpec((nf, cb), lambda i: (0, i + ng))
    return pl.pallas_call(
        _mlp2_pack_body,
        grid=(ng,),
        in_specs=[a_spec, b_spec,
                  _full(w1.shape), _full(b1.shape), _full(w2.shape), _full(b2.shape)],
        out_specs=pl.BlockSpec((cb, 2 * od), lambda i: (i, 0)),
        out_shape=jax.ShapeDtypeStruct((EP, 2 * od), jnp.float32),
    )(xt, xt, w1, b1, w2, b2)


def _msg_body(gs_ref, gd_ref, he_ref, wa_ref, wb_ref, wc_ref, b1_ref,
              w2_ref, b2_ref, o_ref):
    gs = gs_ref[...].astype(jnp.float32)
    gd = gd_ref[...].astype(jnp.float32)
    h = (jnp.dot(gs, wa_ref[...], preferred_element_type=jnp.float32)
         + jnp.dot(gd, wb_ref[...], preferred_element_type=jnp.float32)
         + jnp.dot(he_ref[...], wc_ref[...], preferred_element_type=jnp.float32))
    h = _silu(h + b1_ref[...])
    o_ref[...] = jnp.dot(h, w2_ref[...], preferred_element_type=jnp.float32) + b2_ref[...]


def _g_specs():
    src_spec = pl.BlockSpec((RP, P), lambda i: (i, 0))
    dst_spec = pl.BlockSpec((RP, P), lambda i: (i + E // R_E, 0))
    return src_spec, dst_spec


def _msg(g2, h_edge, mp):
    src_spec, dst_spec = _g_specs()
    return pl.pallas_call(
        _msg_body,
        grid=(E // R_E,),
        in_specs=[src_spec, dst_spec, _rows(RP, P)]
        + [_full(w.shape) for w in mp],
        out_specs=_rows(RP, P),
        out_shape=jax.ShapeDtypeStruct((EP, P), jnp.float32),
    )(g2, g2, h_edge, *mp)


def _node_body(hn_ref, ag_ref, wa_ref, wb_ref, b1_ref, w2_ref, b2_ref,
               g_ref, be_ref, o_ref):
    hn = hn_ref[...]
    h = (jnp.dot(hn, wa_ref[...], preferred_element_type=jnp.float32)
         + jnp.dot(ag_ref[...], wb_ref[...], preferred_element_type=jnp.float32))
    h = _silu(h + b1_ref[...])
    u = jnp.dot(h, w2_ref[...], preferred_element_type=jnp.float32) + b2_ref[...]
    x = hn + u
    m = jnp.mean(x, axis=-1, keepdims=True)
    v = jnp.mean((x - m) ** 2, axis=-1, keepdims=True)
    o_ref[...] = (x - m) * jax.lax.rsqrt(v + 1e-5) * g_ref[...] + be_ref[...]


def _node_update(h_node, agg, up):
    wa, wb, b1, w2, b2, g, be = up
    return pl.pallas_call(
        _node_body,
        grid=(N // R_N,),
        in_specs=[_rows(R_N, D), _rows(R_N, D),
                  _full(wa.shape), _full(wb.shape), _full(b1.shape),
                  _full(w2.shape), _full(b2.shape), _full(g.shape), _full(be.shape)],
        out_specs=_rows(R_N, D),
        out_shape=jax.ShapeDtypeStruct((N, D), jnp.float32),
    )(h_node, agg, wa, wb, b1, w2, b2, g, be)


def _edge_stage_body(gs_ref, gd_ref, he_ref, mavg_ref,
                     ea_ref, eb_ref, ec_ref, e1_ref, ew2_ref, e2_ref,
                     lg_ref, lb_ref,
                     ma_ref, mb_ref, mc_ref, m1_ref, mw2_ref, m2_ref,
                     he_out_ref, msg_out_ref):
    gs = gs_ref[...].astype(jnp.float32)
    gd = gd_ref[...].astype(jnp.float32)
    he = he_ref[...]
    h = (jnp.dot(gs, ea_ref[...], preferred_element_type=jnp.float32)
         + jnp.dot(gd, eb_ref[...], preferred_element_type=jnp.float32)
         + jnp.dot(he, ec_ref[...], preferred_element_type=jnp.float32))
    h = _silu(h + e1_ref[...])
    u = jnp.dot(h, ew2_ref[...], preferred_element_type=jnp.float32) + e2_ref[...]
    he_new = _ln_packed(he + u, lg_ref[...], lb_ref[...], mavg_ref[...])
    he_out_ref[...] = he_new
    m = (jnp.dot(gs, ma_ref[...], preferred_element_type=jnp.float32)
         + jnp.dot(gd, mb_ref[...], preferred_element_type=jnp.float32)
         + jnp.dot(he_new, mc_ref[...], preferred_element_type=jnp.float32))
    m = _silu(m + m1_ref[...])
    msg_out_ref[...] = jnp.dot(m, mw2_ref[...], preferred_element_type=jnp.float32) + m2_ref[...]


def _edge_stage(g2, h_edge, mavg, ep, mp):
    src_spec, dst_spec = _g_specs()
    ws = list(ep) + list(mp)
    return pl.pallas_call(
        _edge_stage_body,
        grid=(E // R_E,),
        in_specs=[src_spec, dst_spec, _rows(RP, P), _full(mavg.shape)]
        + [_full(w.shape) for w in ws],
        out_specs=[_rows(RP, P), _rows(RP, P)],
        out_shape=[jax.ShapeDtypeStruct((EP, P), jnp.float32),
                   jax.ShapeDtypeStruct((EP, P), jnp.float32)],
    )(g2, g2, h_edge, mavg, *ws)


def _final_body(gs_ref, gd_ref, he_ref, mavg_ref, sel_ref,
                ea_ref, eb_ref, ec_ref, e1_ref, ew2_ref, e2_ref,
                lg_ref, lb_ref,
                m1a_ref, m1b_ref, m1c_ref, mb1_ref, m2_ref, mb2_ref, m3_ref, mb3_ref,
                r1a_ref, r1b_ref, r1c_ref, rb1_ref, r2_ref, rb2_ref, r3_ref, rb3_ref,
                merge_ref, risk_ref):
    gs = gs_ref[...].astype(jnp.float32)
    gd = gd_ref[...].astype(jnp.float32)
    he = he_ref[...]
    h = (jnp.dot(gs, ea_ref[...], preferred_element_type=jnp.float32)
         + jnp.dot(gd, eb_ref[...], preferred_element_type=jnp.float32)
         + jnp.dot(he, ec_ref[...], preferred_element_type=jnp.float32))
    h = _silu(h + e1_ref[...])
    u = jnp.dot(h, ew2_ref[...], preferred_element_type=jnp.float32) + e2_ref[...]
    he_new = _ln_packed(he + u, lg_ref[...], lb_ref[...], mavg_ref[...])

    def head(w1a, w1b, w1c, b1, w2, b2, w3, b3):
        h1 = (jnp.dot(gs, w1a, preferred_element_type=jnp.float32)
              + jnp.dot(gd, w1b, preferred_element_type=jnp.float32)
              + jnp.dot(he_new, w1c, preferred_element_type=jnp.float32))
        h1 = _silu(h1 + b1)
        h2 = _silu(jnp.dot(h1, w2, preferred_element_type=jnp.float32) + b2)
        return jnp.dot(h2 * w3, sel_ref[...],
                       preferred_element_type=jnp.float32) + b3[0, 0]

    merge_ref[...] = head(m1a_ref[...], m1b_ref[...], m1c_ref[...], mb1_ref[...],
                          m2_ref[...], mb2_ref[...], m3_ref[...], mb3_ref[...])
    risk_ref[...] = jax.nn.sigmoid(
        head(r1a_ref[...], r1b_ref[...], r1c_ref[...], rb1_ref[...],
             r2_ref[...], rb2_ref[...], r3_ref[...], rb3_ref[...]))


def _final_stage(g2, h_edge, mavg, sel, ep, hp_merge, hp_risk):
    src_spec, dst_spec = _g_specs()
    ws = list(ep) + list(hp_merge) + list(hp_risk)
    out_spec = pl.BlockSpec((RP, 2), lambda i: (i, 0))
    merge, risk = pl.pallas_call(
        _final_body,
        grid=(E // R_E,),
        in_specs=[src_spec, dst_spec, _rows(RP, P), _full(mavg.shape),
                  _full(sel.shape)]
        + [_full(w.shape) for w in ws],
        out_specs=[out_spec, out_spec],
        out_shape=[jax.ShapeDtypeStruct((EP, 2), jnp.float32),
                   jax.ShapeDtypeStruct((EP, 2), jnp.float32)],
    )(g2, g2, h_edge, mavg, sel, *ws)
    return (jnp.concatenate([merge[:, 0], merge[:, 1]]),
            jnp.concatenate([risk[:, 0], risk[:, 1]]))


_NC, _NS = 2, 16
_NW = _NC * _NS
_CH = 128
_GCH = 12512
_CPW = _GCH // _NW
_GPAD = _GCH * _CH
_KB = 8


def _gather_body(table, idx3, out, idx_v, *rest):
    bufs = rest[:_KB]
    gsem, wsem = rest[_KB], rest[_KB + 1]
    w = lax.axis_index("s") * _NC + lax.axis_index("c")
    pltpu.sync_copy(idx3.at[w], idx_v)
    base = w * _CPW
    ngrp = (_CPW + _KB - 1) // _KB

    def grp(g, carry):
        for b in range(_KB):
            j = g * _KB + b

            @pl.when(j < _CPW)
            def _():
                @pl.when(g > 0)
                def _():
                    pltpu.make_async_copy(
                        bufs[b], out.at[pl.ds((base + j - _KB) * _CH, _CH)],
                        wsem.at[b]).wait()
                pltpu.async_copy(table.at[idx_v.at[j]], bufs[b], gsem.at[b])
        for b in range(_KB):
            j = g * _KB + b

            @pl.when(j < _CPW)
            def _():
                pltpu.make_async_copy(table.at[idx_v.at[j]], bufs[b],
                                      gsem.at[b]).wait()
                pltpu.async_copy(bufs[b], out.at[pl.ds((base + j) * _CH, _CH)],
                                 wsem.at[b])
        return carry

    lax.fori_loop(0, ngrp, grp, 0)
    ntail = _CPW - (ngrp - 1) * _KB
    for b in range(_KB):
        j = (ngrp - 1) * _KB + b if b < ntail else (ngrp - 2) * _KB + b
        pltpu.make_async_copy(bufs[b], out.at[pl.ds((base + j) * _CH, _CH)],
                              wsem.at[b]).wait()


def _gather(h_node, idx3):
    mesh = plsc.VectorSubcoreMesh(core_axis_name="c", subcore_axis_name="s")
    return pl.kernel(
        _gather_body,
        mesh=mesh,
        compiler_params=pltpu.CompilerParams(use_tc_tiling_on_sc=False),
        out_type=jax.ShapeDtypeStruct((_GPAD, D), jnp.float32),
        scratch_types=(
            [pltpu.VMEM((_CPW, _CH), jnp.int32)]
            + [pltpu.VMEM((_CH, D), jnp.float32) for _ in range(_KB)]
            + [pltpu.SemaphoreType.DMA((_KB,)), pltpu.SemaphoreType.DMA((_KB,))]
        ),
    )(h_node, idx3)


_HQ = D // 4
_SCH = E // _CH
_SCHP = 6256
_CPT = _SCHP // _NS
_AROW = 51200
_ZB = _AROW // _NS // _CH
_KS = 8
_HCH = _CH // 2


def _scatter_body(msg2, dst2, agg, idx_v, zbuf, acc, *rest):
    vbufs = rest[:_KS]
    lsem, ssem = rest[_KS], rest[_KS + 1]
    c = lax.axis_index("c")
    s = lax.axis_index("s")
    base = s * _CPT
    pltpu.sync_copy(dst2.at[pl.ds(base, _CPT)], idx_v)

    def zb(i, carry):
        zbuf[i, pl.ds(0, 16)] = jnp.zeros((16,), jnp.float32)
        return carry

    lax.fori_loop(0, _CH, zb, 0)

    ngrp = (_CPT + _KS - 1) // _KS

    for half in range(2):
        col0 = c * _HQ + half * 2 * _HQ

        def zg(k, carry):
            pltpu.sync_copy(zbuf, acc.at[pl.ds((s * _ZB + k) * _CH, _CH)])
            return carry

        lax.fori_loop(0, _ZB, zg, 0)
        plsc.subcore_barrier()

        def ld(j, b):
            r0 = (base + j) * _HCH
            even = pltpu.async_copy(
                msg2.at[pl.ds(r0, _HCH), pl.ds(col0, _HQ)],
                vbufs[b].at[pl.ds(0, _HCH)], lsem.at[b])
            odd = pltpu.async_copy(
                msg2.at[pl.ds(r0, _HCH), pl.ds(D + col0, _HQ)],
                vbufs[b].at[pl.ds(_HCH, _HCH)], lsem.at[b])
            return even, odd

        def grp(g, carry):
            for b in range(_KS):
                j = g * _KS + b

                @pl.when((j < _CPT) & (base + j < _SCH))
                def _():
                    @pl.when(g > 0)
                    def _():
                        pltpu.make_async_copy(vbufs[b], acc.at[pl.ds(0, _CH)],
                                              ssem.at[b]).wait()
                    ld(j, b)
            for b in range(_KS):
                j = g * _KS + b

                @pl.when((j < _CPT) & (base + j < _SCH))
                def _():
                    r0 = (base + j) * _HCH
                    pltpu.make_async_copy(
                        msg2.at[pl.ds(r0, _HCH), pl.ds(col0, _HQ)],
                        vbufs[b].at[pl.ds(0, _HCH)], lsem.at[b]).wait()
                    pltpu.make_async_copy(
                        msg2.at[pl.ds(r0, _HCH), pl.ds(D + col0, _HQ)],
                        vbufs[b].at[pl.ds(_HCH, _HCH)], lsem.at[b]).wait()
                    pltpu.async_copy(vbufs[b], acc.at[idx_v.at[j]], ssem.at[b],
                                     add=True)
            return carry

        lax.fori_loop(0, ngrp, grp, 0)
        for b in range(_KS):
            last = (_SCH - 1 - base - b) // _KS

            @pl.when(last >= 0)
            def _():
                pltpu.make_async_copy(vbufs[b], acc.at[pl.ds(0, _CH)],
                                      ssem.at[b]).wait()
        plsc.subcore_barrier()
        nr = N // _NS
        pltpu.sync_copy(acc.at[pl.ds(s * nr, nr)],
                        agg.at[pl.ds(s * nr, nr), pl.ds(col0, _HQ)])
        plsc.subcore_barrier()


def _scatter_add(msg2, dst2):
    mesh = plsc.VectorSubcoreMesh(core_axis_name="c", subcore_axis_name="s")
    return pl.kernel(
        _scatter_body,
        mesh=mesh,
        compiler_params=pltpu.CompilerParams(use_tc_tiling_on_sc=False),
        out_type=jax.ShapeDtypeStruct((N, D), jnp.float32),
        scratch_types=(
            [pltpu.VMEM((_CPT, _CH), jnp.int32),
             pltpu.VMEM((_CH, _HQ), jnp.float32),
             pltpu.VMEM_SHARED((_AROW, _HQ), jnp.float32)]
            + [pltpu.VMEM((_CH, _HQ), jnp.float32) for _ in range(_KS)]
            + [pltpu.SemaphoreType.DMA((_KS,)), pltpu.SemaphoreType.DMA((_KS,))]
        ),
    )(msg2, dst2)


def _bd(w):
    return jnp.kron(jnp.eye(2, dtype=jnp.float32), w)


def _t2(b):
    return jnp.tile(b.reshape(1, -1), (1, 2))


def _split3(w):
    return w[:D], w[D:2 * D], w[2 * D:]


def _prep_node_embed(ps):
    (w1, b1), (w2, b2) = ps
    return w1, b1.reshape(1, -1), w2, b2.reshape(1, -1)


def _prep_edge_embed(ps):
    (w1, b1), (w2, b2) = ps
    return w1, b1.reshape(1, -1), w2, b2.reshape(1, -1)


def _prep_msg(ps):
    (w1, b1), (w2, b2) = ps
    wa, wb, wc = _split3(w1)
    return _bd(wa), _bd(wb), _bd(wc), _t2(b1), _bd(w2), _t2(b2)


def _prep_upd(ps, norm):
    (w1, b1), (w2, b2) = ps
    wa, wb = w1[:D], w1[D:]
    g, be = norm
    return wa, wb, b1.reshape(1, -1), w2, b2.reshape(1, -1), g.reshape(1, -1), be.reshape(1, -1)


def _prep_edge(ps, norm):
    (w1, b1), (w2, b2) = ps
    wa, wb, wc = _split3(w1)
    g, be = norm
    return _bd(wa), _bd(wb), _bd(wc), _t2(b1), _bd(w2), _t2(b2), _t2(g), _t2(be)


def _prep_head(ps):
    (w1, b1), (w2, b2), (w3, b3) = ps
    wa, wb, wc = _split3(w1)
    return (_bd(wa), _bd(wb), _bd(wc), _t2(b1), _bd(w2), _t2(b2),
            _t2(w3.reshape(1, -1)), b3.reshape(1, 1))


def kernel(node_feat, edge_index, edge_feat, params):
    src = edge_index[:, 0]
    dst = edge_index[:, 1]
    def _halves(a):
        return jnp.concatenate([a[:EP].reshape(_SCH, _HCH),
                                a[EP:].reshape(_SCH, _HCH)], axis=1)

    src_p = jnp.stack([src[:EP], src[EP:]], axis=1).reshape(E)
    dst_p = jnp.stack([dst[:EP], dst[EP:]], axis=1).reshape(E)
    idx3 = jnp.concatenate(
        [src_p, dst_p, jnp.zeros((_GPAD - 2 * E,), jnp.int32)]).reshape(_NW, _CPW, _CH)
    dst2 = jnp.concatenate(
        [_halves(dst), jnp.zeros((_SCHP - _SCH, _CH), jnp.int32)])

    mavg = _bd(jnp.full((D, D), 1.0 / D, jnp.float32))
    sel = _bd(jnp.ones((D, 1), jnp.float32))

    ne = _prep_node_embed(params["node_embed"])
    ee = _prep_edge_embed(params["edge_embed"])
    layers = [{
        "msg": _prep_msg(lp["msg"]),
        "upd": _prep_upd(lp["upd"], lp["node_norm"]),
        "edge": _prep_edge(lp["edge_upd"], lp["edge_norm"]),
    } for lp in params["layers"]]
    hp_merge = _prep_head(params["merge_head"])
    hp_risk = _prep_head(params["risk_head"])

    h_node = _mlp2(node_feat, *ne, R_N)
    h_edge = _mlp2_pack(jnp.swapaxes(edge_feat, 0, 1), *ee, CB_EMB)

    g2 = _gather(h_node, idx3).reshape(_GPAD // 2, P)
    msg = _msg(g2, h_edge, layers[0]["msg"])
    for i in range(6):
        agg = _scatter_add(msg, dst2)
        h_node = _node_update(h_node, agg, layers[i]["upd"])
        g2 = _gather(h_node, idx3).reshape(_GPAD // 2, P)
        if i < 5:
            h_edge, msg = _edge_stage(g2, h_edge, mavg,
                                      layers[i]["edge"], layers[i + 1]["msg"])
        else:
            merge, risk = _final_stage(g2, h_edge, mavg, sel,
                                       layers[i]["edge"], hp_merge, hp_risk)
    return (merge, risk)
